# baseline (speedup 1.0000x reference)
import functools

import jax
import jax.numpy as jnp
from jax import lax
from jax.experimental import pallas as pl
from jax.experimental.pallas import tpu as pltpu
from jax.experimental.pallas import tpu_sc as plsc

_N_ENTITIES = 1000000
_DIM = 64
_MAX_LEN = 200
_BATCH = 4096

_ROWS = _BATCH * _MAX_LEN
_NUM_WORKERS = 32
_CHUNK = 256
_ROWS_PER_WORKER = _ROWS // _NUM_WORKERS
_CHUNKS_PER_WORKER = _ROWS_PER_WORKER // _CHUNK
_GROUPS = _CHUNK // 16
_NV = _DIM // 16
_GSLICE = 128
_WIDE = 2 * _DIM
_EPS = 1e-5


def _rsqrt(x):
    i = lax.bitcast_convert_type(x, jnp.int32)
    i = jnp.int32(0x5F3759DF) - (i >> 1)
    y = lax.bitcast_convert_type(i, jnp.float32)
    for _ in range(2):
        y = y * (1.5 - 0.5 * x * y * y)
    return y


_GATHER_DNUMS = lax.GatherDimensionNumbers(
    offset_dims=(), collapsed_slice_dims=(0,), start_index_map=(0,))


def _permute(x, idx):
    return lax.gather(x, idx.reshape(16, 1), _GATHER_DNUMS, (1,),
                      mode=lax.GatherScatterMode.PROMISE_IN_BOUNDS)


def _bitrev3(j):
    return int(f"{j:03b}"[::-1], 2)


def _make_sc_kernel():
    mesh = plsc.VectorSubcoreMesh(core_axis_name="c", subcore_axis_name="s")

    @functools.partial(
        pl.kernel,
        out_type=[
            jax.ShapeDtypeStruct((_ROWS // 2, _WIDE), jnp.float32),
            jax.ShapeDtypeStruct((_ROWS,), jnp.int32),
        ],
        mesh=mesh,
        compiler_params=pltpu.CompilerParams(use_tc_tiling_on_sc=False),
        scratch_types=[
            pltpu.VMEM((_CHUNK,), jnp.int32),
            pltpu.VMEM((_CHUNK,), jnp.int32),
            pltpu.VMEM((_CHUNK,), jnp.int32),
            pltpu.VMEM((_CHUNK,), jnp.int32),
            pltpu.VMEM((_CHUNK, _WIDE), jnp.float32),
            pltpu.VMEM((_CHUNK, _WIDE), jnp.float32),
            pltpu.VMEM((_CHUNK // 2, _WIDE), jnp.float32),
            pltpu.VMEM((_MAX_LEN * _DIM,), jnp.float32),
            pltpu.VMEM((_DIM,), jnp.float32),
            pltpu.VMEM((_DIM,), jnp.float32),
            pltpu.VMEM((_CHUNK,), jnp.int32),
            pltpu.SemaphoreType.DMA,
            pltpu.SemaphoreType.DMA,
        ],
    )
    def body(items_ref, table_ref, pos_ref, gamma_ref, beta_ref,
             out_ref, mask_ref,
             idx0, idx1, hid0, hid1, rows0, rows1, obuf, pos_v,
             gamma_v, beta_v, mask_v, sem0, sem1):
        nc = 2
        wid = lax.axis_index("s") * nc + lax.axis_index("c")
        row0 = wid * _ROWS_PER_WORKER

        pltpu.sync_copy(pos_ref, pos_v)
        pltpu.sync_copy(gamma_ref, gamma_v)
        pltpu.sync_copy(beta_ref, beta_v)

        gvs = [gamma_v[pl.ds(16 * k, 16)] for k in range(_NV)]
        bvs = [beta_v[pl.ds(16 * k, 16)] for k in range(_NV)]

        lane = lax.iota(jnp.int32, 16)
        bcast = [jnp.full((16,), j, dtype=jnp.int32) for j in range(16)]
        xors = (8, 4, 2, 1)
        pconst = {x: lane ^ x for x in xors}
        mconst = {x: (lane & x) == 0 for x in xors}

        def _merge(a, b, xor):
            pa = _permute(a, pconst[xor])
            pb = _permute(b, pconst[xor])
            return jnp.where(mconst[xor], a, pb) + jnp.where(mconst[xor], pa, b)

        def gather_descs(hbuf, rowsbuf, sem):
            return [
                pltpu.make_async_copy(
                    table_ref.at[hbuf.at[pl.ds(j * _GSLICE, _GSLICE)]],
                    rowsbuf.at[pl.ds(j * _GSLICE, _GSLICE)],
                    sem,
                )
                for j in range(_CHUNK // _GSLICE)
            ]

        def fire_gather(c, idxbuf, hbuf, rowsbuf, sem):
            base = row0 + c * _CHUNK
            pltpu.sync_copy(items_ref.at[pl.ds(base, _CHUNK)], idxbuf)
            for g in range(_GROUPS):
                iv = idxbuf[pl.ds(g * 16, 16)]
                hbuf[pl.ds(g * 16, 16)] = iv >> 1
            for d in gather_descs(hbuf, rowsbuf, sem):
                d.start()

        def wait_gather(hbuf, rowsbuf, sem):
            for d in gather_descs(hbuf, rowsbuf, sem):
                d.wait()

        def compute_chunk(c, idxbuf, rowsbuf):
            base = row0 + c * _CHUNK

            @plsc.parallel_loop(0, _GROUPS)
            def group_body(g):
                r0 = g * 16
                iv = idxbuf[pl.ds(r0, 16)]
                mb = iv != 0
                mfv = jnp.where(mb, 1.0, 0.0).astype(jnp.float32)
                mask_v[pl.ds(r0, 16)] = jnp.where(mb, 1, 0).astype(jnp.int32)
                stacks = [[], []]
                for jj in range(8):
                    rows_es = []
                    for h in range(2):
                        r = r0 + 8 * h + jj
                        mf = _permute(mfv, bcast[8 * h + jj])
                        p = (base + r) % _MAX_LEN
                        off = (iv[8 * h + jj] & 1) * _DIM
                        vs = [rowsbuf[r, pl.ds(off + 16 * k, 16)]
                              for k in range(_NV)]
                        vps = [pos_v[pl.ds(p * _DIM + 16 * k, 16)]
                               for k in range(_NV)]
                        es = [(vs[k] + vps[k]) * mf for k in range(_NV)]
                        for k in range(_NV):
                            rowsbuf[r, pl.ds(16 * k, 16)] = es[k]
                        rows_es.append(es)
                    for h in range(2):
                        es = rows_es[h]
                        s4 = (es[0] + es[1]) + (es[2] + es[3])
                        q4 = (es[0] * es[0] + es[1] * es[1]) + \
                             (es[2] * es[2] + es[3] * es[3])
                        item = (0, s4, q4)
                        stack = stacks[h]
                        while stack and stack[-1][0] == item[0]:
                            lvl, ts, tq = stack.pop()
                            xor = 8 >> lvl
                            item = (lvl + 1, _merge(ts, item[1], xor),
                                    _merge(tq, item[2], xor))
                        stack.append(item)
                stats = []
                for h in range(2):
                    (_, t_s, t_q), = stacks[h]
                    t_s = t_s + _permute(t_s, pconst[1])
                    t_q = t_q + _permute(t_q, pconst[1])
                    mean_p = t_s * (1.0 / _DIM)
                    var_p = t_q * (1.0 / _DIM) - mean_p * mean_p
                    rstd_p = _rsqrt(var_p + _EPS)
                    stats.append((mean_p, rstd_p))
                for jj in range(8):
                    for h in range(2):
                        r = r0 + 8 * h + jj
                        mean_p, rstd_p = stats[h]
                        bl = bcast[2 * _bitrev3(jj)]
                        mean_b = _permute(mean_p, bl)
                        rstd_b = _permute(rstd_p, bl)
                        for k in range(_NV):
                            e = rowsbuf[r, pl.ds(16 * k, 16)]
                            o = (e - mean_b) * (rstd_b * gvs[k]) + bvs[k]
                            obuf[r // 2, pl.ds((r % 2) * _DIM + 16 * k, 16)] = o

            pltpu.sync_copy(obuf, out_ref.at[pl.ds((base // 2), _CHUNK // 2)])
            pltpu.sync_copy(mask_v, mask_ref.at[pl.ds(base, _CHUNK)])

        fire_gather(0, idx0, hid0, rows0, sem0)

        def pair_body(i, carry):
            a = 2 * i
            b = 2 * i + 1
            fire_gather(b, idx1, hid1, rows1, sem1)
            wait_gather(hid0, rows0, sem0)
            compute_chunk(a, idx0, rows0)
            nxt = jnp.minimum(a + 2, _CHUNKS_PER_WORKER - 2)
            fire_gather(nxt, idx0, hid0, rows0, sem0)
            wait_gather(hid1, rows1, sem1)
            compute_chunk(b, idx1, rows1)
            return carry

        lax.fori_loop(0, _CHUNKS_PER_WORKER // 2, pair_body, 0)
        wait_gather(hid0, rows0, sem0)

    return body


_sc_kernel = _make_sc_kernel()


def kernel(items, node_table, pos_table, gamma, beta):
    items_flat = items.reshape(-1).astype(jnp.int32)
    table_wide = node_table.reshape(_N_ENTITIES // 2 + 1, _WIDE)
    pos_flat = pos_table.reshape(-1)
    out_wide, mask_i32 = _sc_kernel(items_flat, table_wide, pos_flat,
                                    gamma, beta)
    out = out_wide.reshape(_BATCH, _MAX_LEN, _DIM)
    mask = (mask_i32 != 0).reshape(_BATCH, _MAX_LEN)
    return (out, mask)


# speedup vs baseline: 1.3629x; 1.3629x over previous
import functools

import jax
import jax.numpy as jnp
from jax import lax
from jax.experimental import pallas as pl
from jax.experimental.pallas import tpu as pltpu
from jax.experimental.pallas import tpu_sc as plsc

_N_ENTITIES = 1000000
_DIM = 64
_MAX_LEN = 200
_BATCH = 4096

_ROWS = _BATCH * _MAX_LEN
_NUM_WORKERS = 32
_CHUNK = 2 * _MAX_LEN
_ROWS_PER_WORKER = _ROWS // _NUM_WORKERS
_CHUNKS_PER_WORKER = _ROWS_PER_WORKER // _CHUNK
_GROUPS = _CHUNK // 16
_NV = _DIM // 16
_GSLICE = 128
_EPS = 1e-5


def _rsqrt(x):
    i = lax.bitcast_convert_type(x, jnp.int32)
    i = jnp.int32(0x5F3759DF) - (i >> 1)
    y = lax.bitcast_convert_type(i, jnp.float32)
    for _ in range(2):
        y = y * (1.5 - 0.5 * x * y * y)
    return y


_GATHER_DNUMS = lax.GatherDimensionNumbers(
    offset_dims=(), collapsed_slice_dims=(0,), start_index_map=(0,))


def _permute(x, idx):
    return lax.gather(x, idx.reshape(16, 1), _GATHER_DNUMS, (1,),
                      mode=lax.GatherScatterMode.PROMISE_IN_BOUNDS)


def _bitrev2(j):
    return int(f"{j:02b}"[::-1], 2)


def _make_sc_kernel():
    mesh = plsc.VectorSubcoreMesh(core_axis_name="c", subcore_axis_name="s")

    @functools.partial(
        pl.kernel,
        out_type=[
            jax.ShapeDtypeStruct((_BATCH, _MAX_LEN, _DIM), jnp.float32),
            jax.ShapeDtypeStruct((_ROWS,), jnp.int32),
        ],
        mesh=mesh,
        compiler_params=pltpu.CompilerParams(use_tc_tiling_on_sc=False),
        scratch_types=[
            pltpu.VMEM((_CHUNK,), jnp.int32),
            pltpu.VMEM((_CHUNK,), jnp.int32),
            pltpu.VMEM((_CHUNK, _DIM), jnp.float32),
            pltpu.VMEM((_CHUNK, _DIM), jnp.float32),
            pltpu.VMEM((2, _MAX_LEN, _DIM), jnp.float32),
            pltpu.VMEM((_MAX_LEN * _DIM,), jnp.float32),
            pltpu.VMEM((_DIM,), jnp.float32),
            pltpu.VMEM((_DIM,), jnp.float32),
            pltpu.VMEM((_CHUNK,), jnp.int32),
            pltpu.SemaphoreType.DMA,
            pltpu.SemaphoreType.DMA,
        ],
    )
    def body(items_ref, table_ref, pos_ref, gamma_ref, beta_ref,
             out_ref, mask_ref,
             idx0, idx1, rows0, rows1, obuf, pos_v, gamma_v, beta_v,
             mask_v, sem0, sem1):
        nc = 2
        wid = lax.axis_index("s") * nc + lax.axis_index("c")
        row0 = wid * _ROWS_PER_WORKER
        seq0 = wid * (_ROWS_PER_WORKER // _MAX_LEN)

        pltpu.sync_copy(pos_ref, pos_v)
        pltpu.sync_copy(gamma_ref, gamma_v)
        pltpu.sync_copy(beta_ref, beta_v)

        gvs = [gamma_v[pl.ds(16 * k, 16)] for k in range(_NV)]
        bvs = [beta_v[pl.ds(16 * k, 16)] for k in range(_NV)]

        lane = lax.iota(jnp.int32, 16)
        bcast = [jnp.full((16,), j, dtype=jnp.int32) for j in range(16)]
        xors = (8, 4, 2, 1)
        pconst = {x: lane ^ x for x in xors}
        mconst = {x: (lane & x) == 0 for x in xors}

        def _merge(a, b, xor):
            pa = _permute(a, pconst[xor])
            pb = _permute(b, pconst[xor])
            return jnp.where(mconst[xor], a, pb) + jnp.where(mconst[xor], pa, b)

        _slices = []
        off = 0
        while off < _CHUNK:
            n = min(_GSLICE, _CHUNK - off)
            _slices.append((off, n))
            off += n

        def gather_descs(idxbuf, rowsbuf, sem):
            return [
                pltpu.make_async_copy(
                    table_ref.at[idxbuf.at[pl.ds(o, n)]],
                    rowsbuf.at[pl.ds(o, n)],
                    sem,
                )
                for o, n in _slices
            ]

        def fire_gather(c, idxbuf, rowsbuf, sem):
            base = row0 + c * _CHUNK
            pltpu.sync_copy(items_ref.at[pl.ds(base, _CHUNK)], idxbuf)
            for d in gather_descs(idxbuf, rowsbuf, sem):
                d.start()

        def wait_gather(idxbuf, rowsbuf, sem):
            for d in gather_descs(idxbuf, rowsbuf, sem):
                d.wait()

        def compute_chunk(c, idxbuf, rowsbuf):
            base = row0 + c * _CHUNK

            @plsc.parallel_loop(0, _GROUPS)
            def group_body(g):
                r0 = g * 16
                iv = idxbuf[pl.ds(r0, 16)]
                mb = iv != 0
                mfv = jnp.where(mb, 1.0, 0.0).astype(jnp.float32)
                mask_v[pl.ds(r0, 16)] = jnp.where(mb, 1, 0).astype(jnp.int32)
                for h in range(4):
                    all_es = []
                    stack = []
                    for jj in range(4):
                        j = 4 * h + jj
                        r = r0 + j
                        p = r % _MAX_LEN
                        mf = _permute(mfv, bcast[j])
                        vs = [rowsbuf[r, pl.ds(16 * k, 16)]
                              for k in range(_NV)]
                        vps = [pos_v[pl.ds(p * _DIM + 16 * k, 16)]
                               for k in range(_NV)]
                        es = [(vs[k] + vps[k]) * mf for k in range(_NV)]
                        all_es.append(es)
                        s4 = (es[0] + es[1]) + (es[2] + es[3])
                        q4 = (es[0] * es[0] + es[1] * es[1]) + \
                             (es[2] * es[2] + es[3] * es[3])
                        item = (0, s4, q4)
                        while stack and stack[-1][0] == item[0]:
                            lvl, ts, tq = stack.pop()
                            xor = 8 >> lvl
                            item = (lvl + 1, _merge(ts, item[1], xor),
                                    _merge(tq, item[2], xor))
                        stack.append(item)
                    (_, t_s, t_q), = stack
                    t_s = t_s + _permute(t_s, pconst[2])
                    t_q = t_q + _permute(t_q, pconst[2])
                    t_s = t_s + _permute(t_s, pconst[1])
                    t_q = t_q + _permute(t_q, pconst[1])
                    mean_p = t_s * (1.0 / _DIM)
                    var_p = t_q * (1.0 / _DIM) - mean_p * mean_p
                    rstd_p = _rsqrt(var_p + _EPS)
                    for jj in range(4):
                        j = 4 * h + jj
                        r = r0 + j
                        es = all_es[jj]
                        bl = bcast[4 * _bitrev2(jj)]
                        mean_b = _permute(mean_p, bl)
                        rstd_b = _permute(rstd_p, bl)
                        for k in range(_NV):
                            o = (es[k] - mean_b) * (rstd_b * gvs[k]) + bvs[k]
                            obuf[r // _MAX_LEN, r % _MAX_LEN,
                                 pl.ds(16 * k, 16)] = o

            pltpu.sync_copy(obuf, out_ref.at[pl.ds(seq0 + 2 * c, 2)])
            pltpu.sync_copy(mask_v, mask_ref.at[pl.ds(base, _CHUNK)])

        fire_gather(0, idx0, rows0, sem0)

        def pair_body(i, carry):
            a = 2 * i
            b = 2 * i + 1
            fire_gather(b, idx1, rows1, sem1)
            wait_gather(idx0, rows0, sem0)
            compute_chunk(a, idx0, rows0)
            nxt = jnp.minimum(a + 2, _CHUNKS_PER_WORKER - 2)
            fire_gather(nxt, idx0, rows0, sem0)
            wait_gather(idx1, rows1, sem1)
            compute_chunk(b, idx1, rows1)
            return carry

        lax.fori_loop(0, _CHUNKS_PER_WORKER // 2, pair_body, 0)
        wait_gather(idx0, rows0, sem0)

    return body


_sc_kernel = _make_sc_kernel()


def kernel(items, node_table, pos_table, gamma, beta):
    items_flat = items.reshape(-1).astype(jnp.int32)
    pos_flat = pos_table.reshape(-1)
    out, mask_i32 = _sc_kernel(items_flat, node_table, pos_flat,
                               gamma, beta)
    mask = (mask_i32 != 0).reshape(_BATCH, _MAX_LEN)
    return (out, mask)


# speedup vs baseline: 1.4317x; 1.0505x over previous
import functools

import jax
import jax.numpy as jnp
from jax import lax
from jax.experimental import pallas as pl
from jax.experimental.pallas import tpu as pltpu
from jax.experimental.pallas import tpu_sc as plsc

_N_ENTITIES = 1000000
_DIM = 64
_MAX_LEN = 200
_BATCH = 4096

_ROWS = _BATCH * _MAX_LEN
_NUM_WORKERS = 32
_CHUNK = 2 * _MAX_LEN
_ROWS_PER_WORKER = _ROWS // _NUM_WORKERS
_CHUNKS_PER_WORKER = _ROWS_PER_WORKER // _CHUNK
_GROUPS = _CHUNK // 16
_NV = _DIM // 16
_GSLICE = 128
_EPS = 1e-5


def _rsqrt(x):
    i = lax.bitcast_convert_type(x, jnp.int32)
    i = jnp.int32(0x5F3759DF) - (i >> 1)
    y = lax.bitcast_convert_type(i, jnp.float32)
    for _ in range(2):
        y = y * (1.5 - 0.5 * x * y * y)
    return y


_GATHER_DNUMS = lax.GatherDimensionNumbers(
    offset_dims=(), collapsed_slice_dims=(0,), start_index_map=(0,))


def _permute(x, idx):
    return lax.gather(x, idx.reshape(16, 1), _GATHER_DNUMS, (1,),
                      mode=lax.GatherScatterMode.PROMISE_IN_BOUNDS)


def _bitrev3(j):
    return int(f"{j:03b}"[::-1], 2)


def _make_sc_kernel():
    mesh = plsc.VectorSubcoreMesh(core_axis_name="c", subcore_axis_name="s")

    @functools.partial(
        pl.kernel,
        out_type=[
            jax.ShapeDtypeStruct((_BATCH, _MAX_LEN, _DIM), jnp.float32),
            jax.ShapeDtypeStruct((_ROWS,), jnp.int32),
        ],
        mesh=mesh,
        compiler_params=pltpu.CompilerParams(use_tc_tiling_on_sc=False),
        scratch_types=[
            pltpu.VMEM((_CHUNK,), jnp.int32),
            pltpu.VMEM((_CHUNK,), jnp.int32),
            pltpu.VMEM((_CHUNK, _DIM), jnp.float32),
            pltpu.VMEM((_CHUNK, _DIM), jnp.float32),
            pltpu.VMEM((2, _MAX_LEN, _DIM), jnp.float32),
            pltpu.VMEM((_MAX_LEN * _DIM,), jnp.float32),
            pltpu.VMEM((_DIM,), jnp.float32),
            pltpu.VMEM((_DIM,), jnp.float32),
            pltpu.VMEM((_CHUNK,), jnp.int32),
            pltpu.SemaphoreType.DMA,
            pltpu.SemaphoreType.DMA,
        ],
    )
    def body(items_ref, table_ref, pos_ref, gamma_ref, beta_ref,
             out_ref, mask_ref,
             idx0, idx1, rows0, rows1, obuf, pos_v, gamma_v, beta_v,
             mask_v, sem0, sem1):
        nc = 2
        wid = lax.axis_index("s") * nc + lax.axis_index("c")
        row0 = wid * _ROWS_PER_WORKER
        seq0 = wid * (_ROWS_PER_WORKER // _MAX_LEN)

        pltpu.sync_copy(pos_ref, pos_v)
        pltpu.sync_copy(gamma_ref, gamma_v)
        pltpu.sync_copy(beta_ref, beta_v)

        gvs = [gamma_v[pl.ds(16 * k, 16)] for k in range(_NV)]
        bvs = [beta_v[pl.ds(16 * k, 16)] for k in range(_NV)]

        lane = lax.iota(jnp.int32, 16)
        bcast = [jnp.full((16,), j, dtype=jnp.int32) for j in range(16)]
        xors = (8, 4, 2, 1)
        pconst = {x: lane ^ x for x in xors}
        mconst = {x: (lane & x) == 0 for x in xors}

        def _merge(a, b, xor):
            pa = _permute(a, pconst[xor])
            pb = _permute(b, pconst[xor])
            return jnp.where(mconst[xor], a, pb) + jnp.where(mconst[xor], pa, b)

        _slices = []
        off = 0
        while off < _CHUNK:
            n = min(_GSLICE, _CHUNK - off)
            _slices.append((off, n))
            off += n

        def gather_descs(idxbuf, rowsbuf, sem):
            return [
                pltpu.make_async_copy(
                    table_ref.at[idxbuf.at[pl.ds(o, n)]],
                    rowsbuf.at[pl.ds(o, n)],
                    sem,
                )
                for o, n in _slices
            ]

        def fire_gather(c, idxbuf, rowsbuf, sem):
            base = row0 + c * _CHUNK
            pltpu.sync_copy(items_ref.at[pl.ds(base, _CHUNK)], idxbuf)
            for d in gather_descs(idxbuf, rowsbuf, sem):
                d.start()

        def wait_gather(idxbuf, rowsbuf, sem):
            for d in gather_descs(idxbuf, rowsbuf, sem):
                d.wait()

        def compute_chunk(c, idxbuf, rowsbuf):
            base = row0 + c * _CHUNK

            @plsc.parallel_loop(0, _GROUPS)
            def group_body(g):
                r0 = g * 16
                iv = idxbuf[pl.ds(r0, 16)]
                mb = iv != 0
                mfv = jnp.where(mb, 1.0, 0.0).astype(jnp.float32)
                mask_v[pl.ds(r0, 16)] = jnp.where(mb, 1, 0).astype(jnp.int32)
                for h in range(2):
                    all_es = []
                    stack = []
                    for jj in range(8):
                        j = 8 * h + jj
                        r = r0 + j
                        p = r % _MAX_LEN
                        mf = _permute(mfv, bcast[j])
                        vs = [rowsbuf[r, pl.ds(16 * k, 16)]
                              for k in range(_NV)]
                        vps = [pos_v[pl.ds(p * _DIM + 16 * k, 16)]
                               for k in range(_NV)]
                        es = [(vs[k] + vps[k]) * mf for k in range(_NV)]
                        all_es.append(es)
                        s4 = (es[0] + es[1]) + (es[2] + es[3])
                        q4 = (es[0] * es[0] + es[1] * es[1]) + \
                             (es[2] * es[2] + es[3] * es[3])
                        item = (0, s4, q4)
                        while stack and stack[-1][0] == item[0]:
                            lvl, ts, tq = stack.pop()
                            xor = 8 >> lvl
                            item = (lvl + 1, _merge(ts, item[1], xor),
                                    _merge(tq, item[2], xor))
                        stack.append(item)
                    (_, t_s, t_q), = stack
                    t_s = t_s + _permute(t_s, pconst[1])
                    t_q = t_q + _permute(t_q, pconst[1])
                    mean_p = t_s * (1.0 / _DIM)
                    var_p = t_q * (1.0 / _DIM) - mean_p * mean_p
                    rstd_p = _rsqrt(var_p + _EPS)
                    for jj in range(8):
                        j = 8 * h + jj
                        r = r0 + j
                        es = all_es[jj]
                        bl = bcast[2 * _bitrev3(jj)]
                        mean_b = _permute(mean_p, bl)
                        rstd_b = _permute(rstd_p, bl)
                        for k in range(_NV):
                            o = (es[k] - mean_b) * (rstd_b * gvs[k]) + bvs[k]
                            obuf[r // _MAX_LEN, r % _MAX_LEN,
                                 pl.ds(16 * k, 16)] = o

            pltpu.sync_copy(obuf, out_ref.at[pl.ds(seq0 + 2 * c, 2)])
            pltpu.sync_copy(mask_v, mask_ref.at[pl.ds(base, _CHUNK)])

        fire_gather(0, idx0, rows0, sem0)

        def pair_body(i, carry):
            a = 2 * i
            b = 2 * i + 1
            fire_gather(b, idx1, rows1, sem1)
            wait_gather(idx0, rows0, sem0)
            compute_chunk(a, idx0, rows0)
            nxt = jnp.minimum(a + 2, _CHUNKS_PER_WORKER - 2)
            fire_gather(nxt, idx0, rows0, sem0)
            wait_gather(idx1, rows1, sem1)
            compute_chunk(b, idx1, rows1)
            return carry

        lax.fori_loop(0, _CHUNKS_PER_WORKER // 2, pair_body, 0)
        wait_gather(idx0, rows0, sem0)

    return body


_sc_kernel = _make_sc_kernel()


def kernel(items, node_table, pos_table, gamma, beta):
    items_flat = items.reshape(-1).astype(jnp.int32)
    pos_flat = pos_table.reshape(-1)
    out, mask_i32 = _sc_kernel(items_flat, node_table, pos_flat,
                               gamma, beta)
    mask = (mask_i32 != 0).reshape(_BATCH, _MAX_LEN)
    return (out, mask)
